# baseline (speedup 1.0000x reference)
import math

import jax
import jax.numpy as jnp
from jax import lax
from jax.experimental import pallas as pl
from jax.experimental.pallas import tpu as pltpu
from jax.experimental.pallas import tpu_sc as plsc

N = 100000
H = W = 512
TILE = 16
NTH = NTW = 32
NT = NTH * NTW
FX = FY = 500.0
CX = CY = 256.0

G = 512
STEPS = -(-N // G)
NPAD = STEPS * G
NCORE = 2
NSUB = 16
NWORK = NCORE * NSUB
CHUNK = NPAD // NWORK

_C0 = 0.28209479177387814
_C1 = 0.4886025119029199
_C2 = 1.0925484305920792
_C3 = 0.31539156525252005
_C4 = 0.5462742152960396


def _bt(x):
    return x.astype(jnp.bfloat16).astype(jnp.float32)


def _gauss_body(rwt, mean_ref, qvec_ref, svec_ref, sh_ref, alpha_ref,
                tix_ref, v0_ref, v1_ref, v2_ref, v3_ref):
    step = pl.program_id(0)
    ii = lax.broadcasted_iota(jnp.int32, (1, G), 1).reshape((G,))
    valid = (step * G + ii) < N

    mb = [_bt(mean_ref[:, i]) for i in range(3)]
    rw = lambda i, j: rwt[3 * i + j]
    rwb = lambda i, j: _bt(rwt[3 * i + j])
    p0 = rwb(0, 0) * mb[0] + rwb(0, 1) * mb[1] + rwb(0, 2) * mb[2] + rwt[9]
    p1 = rwb(1, 0) * mb[0] + rwb(1, 1) * mb[1] + rwb(1, 2) * mb[2] + rwt[10]
    p2 = rwb(2, 0) * mb[0] + rwb(2, 1) * mb[1] + rwb(2, 2) * mb[2] + rwt[11]

    zs = jnp.maximum(p2, 1e-4)
    u = FX * p0 / zs + CX
    v = FY * p1 / zs + CY
    mask = (p2 > 0.2) & (u >= 0.0) & (u < float(W)) & (v >= 0.0) & (v < float(H))

    q0 = qvec_ref[:, 0]
    q1 = qvec_ref[:, 1]
    q2 = qvec_ref[:, 2]
    q3 = qvec_ref[:, 3]
    qn = jnp.sqrt(q0 * q0 + q1 * q1 + q2 * q2 + q3 * q3) + 1e-8
    qw, qx, qy, qz = q0 / qn, q1 / qn, q2 / qn, q3 / qn
    r = [[1 - 2 * (qy * qy + qz * qz), 2 * (qx * qy - qw * qz), 2 * (qx * qz + qw * qy)],
         [2 * (qx * qy + qw * qz), 1 - 2 * (qx * qx + qz * qz), 2 * (qy * qz - qw * qx)],
         [2 * (qx * qz - qw * qy), 2 * (qy * qz + qw * qx), 1 - 2 * (qx * qx + qy * qy)]]
    sv = [jnp.exp(svec_ref[:, k]) for k in range(3)]
    mB = [[_bt(r[i][k] * sv[k]) for k in range(3)] for i in range(3)]
    cov = [[mB[i][0] * mB[j][0] + mB[i][1] * mB[j][1] + mB[i][2] * mB[j][2]
            for j in range(3)] for i in range(3)]
    t1 = [[rwb(i, 0) * _bt(cov[0][j]) + rwb(i, 1) * _bt(cov[1][j])
           + rwb(i, 2) * _bt(cov[2][j]) for j in range(3)] for i in range(3)]
    cc = [[_bt(t1[i][0]) * rwb(l, 0) + _bt(t1[i][1]) * rwb(l, 1)
           + _bt(t1[i][2]) * rwb(l, 2) for l in range(3)] for i in range(3)]
    a = FX / zs
    bq = -FX * p0 / (zs * zs)
    c = FY / zs
    dq = -FY * p1 / (zs * zs)
    ab, bqb, cb, dqb = _bt(a), _bt(bq), _bt(c), _bt(dq)
    t20 = [ab * _bt(cc[0][k]) + bqb * _bt(cc[2][k]) for k in range(3)]
    t21 = [cb * _bt(cc[1][k]) + dqb * _bt(cc[2][k]) for k in range(3)]
    v00 = _bt(t20[0]) * ab + _bt(t20[2]) * bqb + 0.3
    v01 = _bt(t20[1]) * cb + _bt(t20[2]) * dqb
    v10 = _bt(t21[0]) * ab + _bt(t21[2]) * bqb
    v11 = _bt(t21[1]) * cb + _bt(t21[2]) * dqb + 0.3
    det = v00 * v11 - v01 * v10
    area = 2.0 * math.pi * jnp.sqrt(jnp.maximum(det, 1e-8))
    alpha = jax.nn.sigmoid(alpha_ref[...])
    wsp = alpha * jnp.clip(area / (TILE * TILE), 0.0, 1.0)
    wsp = jnp.where(valid & mask, wsp, 0.0)

    nrm = jnp.sqrt(p0 * p0 + p1 * p1 + p2 * p2) + 1e-8
    dx, dy, dz = p0 / nrm, p1 / nrm, p2 / nrm
    bs = (jnp.full_like(dx, _C0),
          -_C1 * dy, _C1 * dz, -_C1 * dx,
          _C2 * dx * dy, -_C2 * dy * dz,
          _C3 * (2 * dz * dz - dx * dx - dy * dy),
          -_C2 * dx * dz, _C4 * (dx * dx - dy * dy))
    bsb = [_bt(x) for x in bs]
    wc = []
    for ch in range(3):
        acc = _bt(sh_ref[:, ch, 0]) * bsb[0]
        for k in range(1, 9):
            acc = acc + _bt(sh_ref[:, ch, k]) * bsb[k]
        wc.append(jnp.where(valid, wsp * jax.nn.sigmoid(acc), 0.0))

    ti = jnp.clip(jnp.floor(v / TILE).astype(jnp.int32), 0, NTH - 1)
    tj = jnp.clip(jnp.floor(u / TILE).astype(jnp.int32), 0, NTW - 1)
    tix = jnp.where(valid, ti * NTW + tj, 0)
    tix = jnp.clip(tix, 0, NT - 1)

    tix_ref[...] = tix
    v0_ref[...] = wsp
    v1_ref[...] = wc[0]
    v2_ref[...] = wc[1]
    v3_ref[...] = wc[2]


def _gauss_call(rwt, mean, qvec, svec_raw, sh_coeffs, alpha_raw):
    return pl.pallas_call(
        _gauss_body,
        grid=(STEPS,),
        in_specs=[
            pl.BlockSpec(memory_space=pltpu.SMEM),
            pl.BlockSpec((G, 3), lambda s: (s, 0)),
            pl.BlockSpec((G, 4), lambda s: (s, 0)),
            pl.BlockSpec((G, 3), lambda s: (s, 0)),
            pl.BlockSpec((G, 3, 9), lambda s: (s, 0, 0)),
            pl.BlockSpec((G,), lambda s: (s,)),
        ],
        out_specs=[pl.BlockSpec((G,), lambda s: (s,))] * 5,
        out_shape=[jax.ShapeDtypeStruct((NPAD,), jnp.int32)]
        + [jax.ShapeDtypeStruct((NPAD,), jnp.float32)] * 4,
    )(rwt, mean, qvec, svec_raw, sh_coeffs, alpha_raw)


def _sc_scatter_body(tix_hbm, v0_hbm, v1_hbm, v2_hbm, v3_hbm, out_hbm,
                     a0, a1, a2, a3, idx_v, val_v, zero_v):
    cid = lax.axis_index("c")
    sid = lax.axis_index("s")
    accs = (a0, a1, a2, a3)
    for j in range(4):
        zero_v[pl.ds(j * 16, 16)] = jnp.zeros((16,), jnp.float32)
    for acc in accs:
        pltpu.sync_copy(zero_v, acc.at[pl.ds(sid * 64, 64)])
    plsc.subcore_barrier()
    wid = cid * NSUB + sid
    base = wid * CHUNK
    pltpu.sync_copy(tix_hbm.at[pl.ds(base, CHUNK)], idx_v)
    for ch, v_hbm in enumerate((v0_hbm, v1_hbm, v2_hbm, v3_hbm)):
        pltpu.sync_copy(v_hbm.at[pl.ds(base, CHUNK)], val_v)
        pltpu.sync_copy(val_v, accs[ch].at[idx_v], add=True)
    plsc.subcore_barrier()
    for ch in range(4):
        @pl.when(sid == ch)
        def _write(ch=ch):
            pltpu.sync_copy(accs[ch], out_hbm.at[cid, ch])


def _sc_call(tix, v0, v1, v2, v3):
    return pl.kernel(
        _sc_scatter_body,
        out_type=jax.ShapeDtypeStruct((NCORE, 4, NT), jnp.float32),
        mesh=plsc.VectorSubcoreMesh(core_axis_name="c", subcore_axis_name="s"),
        scratch_types=[
            pltpu.VMEM_SHARED((NT,), jnp.float32),
            pltpu.VMEM_SHARED((NT,), jnp.float32),
            pltpu.VMEM_SHARED((NT,), jnp.float32),
            pltpu.VMEM_SHARED((NT,), jnp.float32),
            pltpu.VMEM((CHUNK,), jnp.int32),
            pltpu.VMEM((CHUNK,), jnp.float32),
            pltpu.VMEM((64,), jnp.float32),
        ],
    )(tix, v0, v1, v2, v3)


def _finish_body(parts_ref, img_ref):
    acc = parts_ref[0] + parts_ref[1]
    deni = 1.0 / (acc[0] + 1e-6)
    colc = lax.broadcasted_iota(jnp.int32, (NTW, W * 3), 1)
    rowt = lax.broadcasted_iota(jnp.int32, (NTW, W * 3), 0)
    q = jnp.zeros((NTH, W * 3), jnp.float32)
    for ch in range(3):
        rgb = acc[ch + 1] * deni
        sel = ((colc % 3 == ch) & (colc // (3 * TILE) == rowt)
               ).astype(jnp.float32)
        q = q + jnp.dot(rgb, sel, precision=lax.Precision.HIGHEST,
                        preferred_element_type=jnp.float32)
    prow = (lax.broadcasted_iota(jnp.int32, (H, NTH), 0) // TILE ==
            lax.broadcasted_iota(jnp.int32, (H, NTH), 1)).astype(jnp.float32)
    img_ref[...] = jnp.dot(prow, q, precision=lax.Precision.HIGHEST,
                           preferred_element_type=jnp.float32)


def _finish_call(parts):
    return pl.pallas_call(
        _finish_body,
        out_shape=jax.ShapeDtypeStruct((H, W * 3), jnp.float32),
    )(parts)


def kernel(mean, qvec, svec_raw, sh_coeffs, alpha_raw, c2w):
    rwm = c2w[:3, :3].T
    t = -rwm @ c2w[:3, 3]
    rwt = jnp.concatenate([rwm.reshape(-1), t,
                           jnp.zeros((4,), jnp.float32)]).astype(jnp.float32)
    tix, v0, v1, v2, v3 = _gauss_call(rwt, mean, qvec, svec_raw,
                                      sh_coeffs, alpha_raw)
    parts = _sc_call(tix, v0, v1, v2, v3)
    img = _finish_call(parts.reshape(NCORE, 4, NTH, NTW))
    return img.reshape(H, W, 3)


# speedup vs baseline: 18.3972x; 18.3972x over previous
import math

import jax
import jax.numpy as jnp
from jax import lax
from jax.experimental import pallas as pl
from jax.experimental.pallas import tpu as pltpu
from jax.experimental.pallas import tpu_sc as plsc

N = 100000
H = W = 512
TILE = 16
NTH = NTW = 32
NT = NTH * NTW
FX = FY = 500.0
CX = CY = 256.0

GS = 32
G = GS * 128
STEPS = -(-N // G)
NPAD = STEPS * G
NROW = NPAD // 128
NF = 38
NCORE = 2
NSUB = 16
NWORK = NCORE * NSUB
CHUNK = NPAD // NWORK

_C0 = 0.28209479177387814
_C1 = 0.4886025119029199
_C2 = 1.0925484305920792
_C3 = 0.31539156525252005
_C4 = 0.5462742152960396


def _bt(x):
    return x.astype(jnp.bfloat16).astype(jnp.float32)


def _gauss_body(rwt, pk_ref, tix_ref, v0_ref, v1_ref, v2_ref, v3_ref):
    step = pl.program_id(0)
    ii = (lax.broadcasted_iota(jnp.int32, (GS, 128), 0) * 128
          + lax.broadcasted_iota(jnp.int32, (GS, 128), 1))
    valid = (step * G + ii) < N

    mb = [_bt(pk_ref[i]) for i in range(3)]
    rwb = lambda i, j: _bt(rwt[3 * i + j])
    p0 = rwb(0, 0) * mb[0] + rwb(0, 1) * mb[1] + rwb(0, 2) * mb[2] + rwt[9]
    p1 = rwb(1, 0) * mb[0] + rwb(1, 1) * mb[1] + rwb(1, 2) * mb[2] + rwt[10]
    p2 = rwb(2, 0) * mb[0] + rwb(2, 1) * mb[1] + rwb(2, 2) * mb[2] + rwt[11]

    zs = jnp.maximum(p2, 1e-4)
    u = FX * p0 / zs + CX
    v = FY * p1 / zs + CY
    mask = (p2 > 0.2) & (u >= 0.0) & (u < float(W)) & (v >= 0.0) & (v < float(H))

    q0 = pk_ref[3]
    q1 = pk_ref[4]
    q2 = pk_ref[5]
    q3 = pk_ref[6]
    qn = jnp.sqrt(q0 * q0 + q1 * q1 + q2 * q2 + q3 * q3) + 1e-8
    qw, qx, qy, qz = q0 / qn, q1 / qn, q2 / qn, q3 / qn
    r = [[1 - 2 * (qy * qy + qz * qz), 2 * (qx * qy - qw * qz), 2 * (qx * qz + qw * qy)],
         [2 * (qx * qy + qw * qz), 1 - 2 * (qx * qx + qz * qz), 2 * (qy * qz - qw * qx)],
         [2 * (qx * qz - qw * qy), 2 * (qy * qz + qw * qx), 1 - 2 * (qx * qx + qy * qy)]]
    sv = [jnp.exp(pk_ref[7 + k]) for k in range(3)]
    mB = [[_bt(r[i][k] * sv[k]) for k in range(3)] for i in range(3)]
    cov = [[mB[i][0] * mB[j][0] + mB[i][1] * mB[j][1] + mB[i][2] * mB[j][2]
            for j in range(3)] for i in range(3)]
    t1 = [[rwb(i, 0) * _bt(cov[0][j]) + rwb(i, 1) * _bt(cov[1][j])
           + rwb(i, 2) * _bt(cov[2][j]) for j in range(3)] for i in range(3)]
    cc = [[_bt(t1[i][0]) * rwb(l, 0) + _bt(t1[i][1]) * rwb(l, 1)
           + _bt(t1[i][2]) * rwb(l, 2) for l in range(3)] for i in range(3)]
    a = FX / zs
    bq = -FX * p0 / (zs * zs)
    c = FY / zs
    dq = -FY * p1 / (zs * zs)
    ab, bqb, cb, dqb = _bt(a), _bt(bq), _bt(c), _bt(dq)
    t20 = [ab * _bt(cc[0][k]) + bqb * _bt(cc[2][k]) for k in range(3)]
    t21 = [cb * _bt(cc[1][k]) + dqb * _bt(cc[2][k]) for k in range(3)]
    v00 = _bt(t20[0]) * ab + _bt(t20[2]) * bqb + 0.3
    v01 = _bt(t20[1]) * cb + _bt(t20[2]) * dqb
    v10 = _bt(t21[0]) * ab + _bt(t21[2]) * bqb
    v11 = _bt(t21[1]) * cb + _bt(t21[2]) * dqb + 0.3
    det = v00 * v11 - v01 * v10
    area = 2.0 * math.pi * jnp.sqrt(jnp.maximum(det, 1e-8))
    alpha = jax.nn.sigmoid(pk_ref[37])
    wsp = alpha * jnp.clip(area / (TILE * TILE), 0.0, 1.0)
    wsp = jnp.where(valid & mask, wsp, 0.0)

    nrm = jnp.sqrt(p0 * p0 + p1 * p1 + p2 * p2) + 1e-8
    dx, dy, dz = p0 / nrm, p1 / nrm, p2 / nrm
    bs = (jnp.full_like(dx, _C0),
          -_C1 * dy, _C1 * dz, -_C1 * dx,
          _C2 * dx * dy, -_C2 * dy * dz,
          _C3 * (2 * dz * dz - dx * dx - dy * dy),
          -_C2 * dx * dz, _C4 * (dx * dx - dy * dy))
    bsb = [_bt(x) for x in bs]
    wc = []
    for ch in range(3):
        acc = _bt(pk_ref[10 + ch * 9]) * bsb[0]
        for k in range(1, 9):
            acc = acc + _bt(pk_ref[10 + ch * 9 + k]) * bsb[k]
        wc.append(jnp.where(valid, wsp * jax.nn.sigmoid(acc), 0.0))

    ti = jnp.clip(jnp.floor(v / TILE).astype(jnp.int32), 0, NTH - 1)
    tj = jnp.clip(jnp.floor(u / TILE).astype(jnp.int32), 0, NTW - 1)
    tix = jnp.where(valid, ti * NTW + tj, 0)
    tix = jnp.clip(tix, 0, NT - 1)

    tix_ref[...] = tix
    v0_ref[...] = wsp
    v1_ref[...] = wc[0]
    v2_ref[...] = wc[1]
    v3_ref[...] = wc[2]


def _gauss_call(rwt, packed):
    return pl.pallas_call(
        _gauss_body,
        grid=(STEPS,),
        in_specs=[
            pl.BlockSpec(memory_space=pltpu.SMEM),
            pl.BlockSpec((NF, GS, 128), lambda s: (0, s, 0)),
        ],
        out_specs=[pl.BlockSpec((GS, 128), lambda s: (s, 0))] * 5,
        out_shape=[jax.ShapeDtypeStruct((NROW, 128), jnp.int32)]
        + [jax.ShapeDtypeStruct((NROW, 128), jnp.float32)] * 4,
    )(rwt, packed)


def _sc_scatter_body(tix_hbm, v0_hbm, v1_hbm, v2_hbm, v3_hbm, out_hbm,
                     a0, a1, a2, a3, idx_v, val_v, zero_v):
    cid = lax.axis_index("c")
    sid = lax.axis_index("s")
    accs = (a0, a1, a2, a3)
    for j in range(4):
        zero_v[pl.ds(j * 16, 16)] = jnp.zeros((16,), jnp.float32)
    for acc in accs:
        pltpu.sync_copy(zero_v, acc.at[pl.ds(sid * 64, 64)])
    plsc.subcore_barrier()
    wid = cid * NSUB + sid
    base = wid * CHUNK
    pltpu.sync_copy(tix_hbm.at[pl.ds(base, CHUNK)], idx_v)
    for ch, v_hbm in enumerate((v0_hbm, v1_hbm, v2_hbm, v3_hbm)):
        pltpu.sync_copy(v_hbm.at[pl.ds(base, CHUNK)], val_v)
        pltpu.sync_copy(val_v, accs[ch].at[idx_v], add=True)
    plsc.subcore_barrier()
    for ch in range(4):
        @pl.when(sid == ch)
        def _write(ch=ch):
            pltpu.sync_copy(accs[ch], out_hbm.at[cid, ch])


def _sc_call(tix, v0, v1, v2, v3):
    return pl.kernel(
        _sc_scatter_body,
        out_type=jax.ShapeDtypeStruct((NCORE, 4, NT), jnp.float32),
        mesh=plsc.VectorSubcoreMesh(core_axis_name="c", subcore_axis_name="s"),
        scratch_types=[
            pltpu.VMEM_SHARED((NT,), jnp.float32),
            pltpu.VMEM_SHARED((NT,), jnp.float32),
            pltpu.VMEM_SHARED((NT,), jnp.float32),
            pltpu.VMEM_SHARED((NT,), jnp.float32),
            pltpu.VMEM((CHUNK,), jnp.int32),
            pltpu.VMEM((CHUNK,), jnp.float32),
            pltpu.VMEM((64,), jnp.float32),
        ],
    )(tix, v0, v1, v2, v3)


def _finish_body(parts_ref, img_ref):
    acc = parts_ref[0] + parts_ref[1]
    deni = 1.0 / (acc[0] + 1e-6)
    colc = lax.broadcasted_iota(jnp.int32, (NTW, W * 3), 1)
    rowt = lax.broadcasted_iota(jnp.int32, (NTW, W * 3), 0)
    q = jnp.zeros((NTH, W * 3), jnp.float32)
    for ch in range(3):
        rgb = acc[ch + 1] * deni
        sel = ((colc % 3 == ch) & (colc // (3 * TILE) == rowt)
               ).astype(jnp.float32)
        q = q + jnp.dot(rgb, sel, precision=lax.Precision.HIGHEST,
                        preferred_element_type=jnp.float32)
    prow = (lax.broadcasted_iota(jnp.int32, (H, NTH), 0) // TILE ==
            lax.broadcasted_iota(jnp.int32, (H, NTH), 1)).astype(jnp.float32)
    img_ref[...] = jnp.dot(prow, q, precision=lax.Precision.HIGHEST,
                           preferred_element_type=jnp.float32)


def _finish_call(parts):
    return pl.pallas_call(
        _finish_body,
        out_shape=jax.ShapeDtypeStruct((H, W * 3), jnp.float32),
    )(parts)


def kernel(mean, qvec, svec_raw, sh_coeffs, alpha_raw, c2w):
    rwm = c2w[:3, :3].T
    t = -rwm @ c2w[:3, 3]
    rwt = jnp.concatenate([rwm.reshape(-1), t,
                           jnp.zeros((4,), jnp.float32)]).astype(jnp.float32)
    packed = jnp.concatenate([
        mean.T, qvec.T, svec_raw.T,
        sh_coeffs.reshape(N, 3 * 9).T,
        alpha_raw[None, :],
    ], axis=0)
    packed = jnp.pad(packed, ((0, 0), (0, NPAD - N))).reshape(NF, NROW, 128)
    tix, v0, v1, v2, v3 = _gauss_call(rwt, packed)
    parts = _sc_call(tix.reshape(NPAD), v0.reshape(NPAD), v1.reshape(NPAD),
                     v2.reshape(NPAD), v3.reshape(NPAD))
    img = _finish_call(parts.reshape(NCORE, 4, NTH, NTW))
    return img.reshape(H, W, 3)


# speedup vs baseline: 45.2260x; 2.4583x over previous
import math

import jax
import jax.numpy as jnp
from jax import lax
from jax.experimental import pallas as pl
from jax.experimental.pallas import tpu as pltpu
from jax.experimental.pallas import tpu_sc as plsc

N = 100000
H = W = 512
TILE = 16
NTH = NTW = 32
NT = NTH * NTW
FX = FY = 500.0
CX = CY = 256.0

GS = 32
G = GS * 128
STEPS = -(-N // G)
NPAD = STEPS * G
NROW = NPAD // 128
NF = 38
NCORE = 2
NSUB = 16
NWORK = NCORE * NSUB
CHUNK = NPAD // NWORK

_C0 = 0.28209479177387814
_C1 = 0.4886025119029199
_C2 = 1.0925484305920792
_C3 = 0.31539156525252005
_C4 = 0.5462742152960396


def _bt(x):
    return x.astype(jnp.bfloat16).astype(jnp.float32)


def _gauss_body(rwt, pb_ref, pf_ref, tix_ref, v0_ref, v1_ref, v2_ref, v3_ref):
    step = pl.program_id(0)
    ii = (lax.broadcasted_iota(jnp.int32, (GS, 128), 0) * 128
          + lax.broadcasted_iota(jnp.int32, (GS, 128), 1))
    valid = (step * G + ii) < N

    mb = [pb_ref[i].astype(jnp.float32) for i in range(3)]
    rwb = lambda i, j: _bt(rwt[3 * i + j])
    p0 = rwb(0, 0) * mb[0] + rwb(0, 1) * mb[1] + rwb(0, 2) * mb[2] + rwt[9]
    p1 = rwb(1, 0) * mb[0] + rwb(1, 1) * mb[1] + rwb(1, 2) * mb[2] + rwt[10]
    p2 = rwb(2, 0) * mb[0] + rwb(2, 1) * mb[1] + rwb(2, 2) * mb[2] + rwt[11]

    zs = jnp.maximum(p2, 1e-4)
    u = FX * p0 / zs + CX
    v = FY * p1 / zs + CY
    mask = (p2 > 0.2) & (u >= 0.0) & (u < float(W)) & (v >= 0.0) & (v < float(H))

    q0 = pf_ref[0]
    q1 = pf_ref[1]
    q2 = pf_ref[2]
    q3 = pf_ref[3]
    qn = jnp.sqrt(q0 * q0 + q1 * q1 + q2 * q2 + q3 * q3) + 1e-8
    qw, qx, qy, qz = q0 / qn, q1 / qn, q2 / qn, q3 / qn
    r = [[1 - 2 * (qy * qy + qz * qz), 2 * (qx * qy - qw * qz), 2 * (qx * qz + qw * qy)],
         [2 * (qx * qy + qw * qz), 1 - 2 * (qx * qx + qz * qz), 2 * (qy * qz - qw * qx)],
         [2 * (qx * qz - qw * qy), 2 * (qy * qz + qw * qx), 1 - 2 * (qx * qx + qy * qy)]]
    sv = [jnp.exp(pf_ref[4 + k]) for k in range(3)]
    mB = [[_bt(r[i][k] * sv[k]) for k in range(3)] for i in range(3)]
    cov = [[mB[i][0] * mB[j][0] + mB[i][1] * mB[j][1] + mB[i][2] * mB[j][2]
            for j in range(3)] for i in range(3)]
    t1 = [[rwb(i, 0) * _bt(cov[0][j]) + rwb(i, 1) * _bt(cov[1][j])
           + rwb(i, 2) * _bt(cov[2][j]) for j in range(3)] for i in range(3)]
    cc = [[_bt(t1[i][0]) * rwb(l, 0) + _bt(t1[i][1]) * rwb(l, 1)
           + _bt(t1[i][2]) * rwb(l, 2) for l in range(3)] for i in range(3)]
    a = FX / zs
    bq = -FX * p0 / (zs * zs)
    c = FY / zs
    dq = -FY * p1 / (zs * zs)
    ab, bqb, cb, dqb = _bt(a), _bt(bq), _bt(c), _bt(dq)
    t20 = [ab * _bt(cc[0][k]) + bqb * _bt(cc[2][k]) for k in range(3)]
    t21 = [cb * _bt(cc[1][k]) + dqb * _bt(cc[2][k]) for k in range(3)]
    v00 = _bt(t20[0]) * ab + _bt(t20[2]) * bqb + 0.3
    v01 = _bt(t20[1]) * cb + _bt(t20[2]) * dqb
    v10 = _bt(t21[0]) * ab + _bt(t21[2]) * bqb
    v11 = _bt(t21[1]) * cb + _bt(t21[2]) * dqb + 0.3
    det = v00 * v11 - v01 * v10
    area = 2.0 * math.pi * jnp.sqrt(jnp.maximum(det, 1e-8))
    alpha = jax.nn.sigmoid(pf_ref[7])
    wsp = alpha * jnp.clip(area / (TILE * TILE), 0.0, 1.0)
    wsp = jnp.where(valid & mask, wsp, 0.0)

    nrm = jnp.sqrt(p0 * p0 + p1 * p1 + p2 * p2) + 1e-8
    dx, dy, dz = p0 / nrm, p1 / nrm, p2 / nrm
    bs = (jnp.full_like(dx, _C0),
          -_C1 * dy, _C1 * dz, -_C1 * dx,
          _C2 * dx * dy, -_C2 * dy * dz,
          _C3 * (2 * dz * dz - dx * dx - dy * dy),
          -_C2 * dx * dz, _C4 * (dx * dx - dy * dy))
    bsb = [_bt(x) for x in bs]
    wc = []
    for ch in range(3):
        acc = pb_ref[3 + ch * 9].astype(jnp.float32) * bsb[0]
        for k in range(1, 9):
            acc = acc + pb_ref[3 + ch * 9 + k].astype(jnp.float32) * bsb[k]
        wc.append(jnp.where(valid, wsp * jax.nn.sigmoid(acc), 0.0))

    ti = jnp.clip(jnp.floor(v / TILE).astype(jnp.int32), 0, NTH - 1)
    tj = jnp.clip(jnp.floor(u / TILE).astype(jnp.int32), 0, NTW - 1)
    tix = jnp.where(valid, ti * NTW + tj, 0)
    tix = jnp.clip(tix, 0, NT - 1)

    tix_ref[...] = tix
    v0_ref[...] = wsp
    v1_ref[...] = wc[0]
    v2_ref[...] = wc[1]
    v3_ref[...] = wc[2]


def _gauss_call(rwt, pb, pf):
    return pl.pallas_call(
        _gauss_body,
        grid=(STEPS,),
        in_specs=[
            pl.BlockSpec(memory_space=pltpu.SMEM),
            pl.BlockSpec((30, GS, 128), lambda s: (0, s, 0)),
            pl.BlockSpec((8, GS, 128), lambda s: (0, s, 0)),
        ],
        out_specs=[pl.BlockSpec((GS, 128), lambda s: (s, 0))] * 5,
        out_shape=[jax.ShapeDtypeStruct((NROW, 128), jnp.int32)]
        + [jax.ShapeDtypeStruct((NROW, 128), jnp.float32)] * 4,
    )(rwt, pb, pf)


def _sc_scatter_body(tix_hbm, v0_hbm, v1_hbm, v2_hbm, v3_hbm, out_hbm,
                     a0, a1, a2, a3, idx_v, val_v, zero_v):
    cid = lax.axis_index("c")
    sid = lax.axis_index("s")
    accs = (a0, a1, a2, a3)
    for j in range(4):
        zero_v[pl.ds(j * 16, 16)] = jnp.zeros((16,), jnp.float32)
    for acc in accs:
        pltpu.sync_copy(zero_v, acc.at[pl.ds(sid * 64, 64)])
    plsc.subcore_barrier()
    wid = cid * NSUB + sid
    base = wid * CHUNK
    pltpu.sync_copy(tix_hbm.at[pl.ds(base, CHUNK)], idx_v)
    for ch, v_hbm in enumerate((v0_hbm, v1_hbm, v2_hbm, v3_hbm)):
        pltpu.sync_copy(v_hbm.at[pl.ds(base, CHUNK)], val_v)
        pltpu.sync_copy(val_v, accs[ch].at[idx_v], add=True)
    plsc.subcore_barrier()
    for ch in range(4):
        @pl.when(sid == ch)
        def _write(ch=ch):
            pltpu.sync_copy(accs[ch], out_hbm.at[cid, ch])


def _sc_call(tix, v0, v1, v2, v3):
    return pl.kernel(
        _sc_scatter_body,
        out_type=jax.ShapeDtypeStruct((NCORE, 4, NT), jnp.float32),
        mesh=plsc.VectorSubcoreMesh(core_axis_name="c", subcore_axis_name="s"),
        scratch_types=[
            pltpu.VMEM_SHARED((NT,), jnp.float32),
            pltpu.VMEM_SHARED((NT,), jnp.float32),
            pltpu.VMEM_SHARED((NT,), jnp.float32),
            pltpu.VMEM_SHARED((NT,), jnp.float32),
            pltpu.VMEM((CHUNK,), jnp.int32),
            pltpu.VMEM((CHUNK,), jnp.float32),
            pltpu.VMEM((64,), jnp.float32),
        ],
    )(tix, v0, v1, v2, v3)


def _finish_body(parts_ref, img_ref):
    acc = parts_ref[0] + parts_ref[1]
    deni = 1.0 / (acc[0] + 1e-6)
    colc = lax.broadcasted_iota(jnp.int32, (NTW, W * 3), 1)
    rowt = lax.broadcasted_iota(jnp.int32, (NTW, W * 3), 0)
    q = jnp.zeros((NTH, W * 3), jnp.float32)
    for ch in range(3):
        rgb = acc[ch + 1] * deni
        sel = ((colc % 3 == ch) & (colc // (3 * TILE) == rowt)
               ).astype(jnp.float32)
        q = q + jnp.dot(rgb, sel, precision=lax.Precision.HIGHEST,
                        preferred_element_type=jnp.float32)
    prow = (lax.broadcasted_iota(jnp.int32, (H, NTH), 0) // TILE ==
            lax.broadcasted_iota(jnp.int32, (H, NTH), 1)).astype(jnp.float32)
    img_ref[...] = jnp.dot(prow, q, precision=lax.Precision.HIGHEST,
                           preferred_element_type=jnp.float32)


def _finish_call(parts):
    return pl.pallas_call(
        _finish_body,
        out_shape=jax.ShapeDtypeStruct((H, W * 3), jnp.float32),
    )(parts)


def kernel(mean, qvec, svec_raw, sh_coeffs, alpha_raw, c2w):
    rwm = c2w[:3, :3].T
    t = -rwm @ c2w[:3, 3]
    rwt = jnp.concatenate([rwm.reshape(-1), t,
                           jnp.zeros((4,), jnp.float32)]).astype(jnp.float32)
    catb = jnp.concatenate([
        mean, sh_coeffs.reshape(N, 3 * 9),
    ], axis=1).astype(jnp.bfloat16)
    catf = jnp.concatenate([
        qvec, svec_raw, alpha_raw[:, None],
    ], axis=1)
    pb = jnp.pad(catb.T, ((0, 0), (0, NPAD - N))).reshape(30, NROW, 128)
    pf = jnp.pad(catf.T, ((0, 0), (0, NPAD - N))).reshape(8, NROW, 128)
    tix, v0, v1, v2, v3 = _gauss_call(rwt, pb, pf)
    parts = _sc_call(tix.reshape(NPAD), v0.reshape(NPAD), v1.reshape(NPAD),
                     v2.reshape(NPAD), v3.reshape(NPAD))
    img = _finish_call(parts.reshape(NCORE, 4, NTH, NTW))
    return img.reshape(H, W, 3)
